# baseline (speedup 1.0000x reference)
import functools

import jax
import jax.numpy as jnp
from jax.experimental import pallas as pl

_BLK = 1000
_HBLK = 2048


def _layer_kernel(s0_ref, s1_ref, s2_ref, rinv_ref, xdst_ref, wl_ref, wrm_ref,
                  bm_ref, out_ref, *, relu):
    r = rinv_ref[...]
    acc = jnp.dot(xdst_ref[...], wrm_ref[...],
                  preferred_element_type=jnp.float32)
    acc += jnp.dot(s0_ref[...] * r[:, 0:1], wl_ref[0],
                   preferred_element_type=jnp.float32)
    acc += jnp.dot(s1_ref[...] * r[:, 1:2], wl_ref[1],
                   preferred_element_type=jnp.float32)
    acc += jnp.dot(s2_ref[...] * r[:, 2:3], wl_ref[2],
                   preferred_element_type=jnp.float32)
    acc += bm_ref[...]
    if relu:
        acc = jnp.maximum(acc, 0.0)
    out_ref[...] = acc


def _layer_call(s0, s1, s2, rinv, xdst, wl, wrm, bm, relu):
    n, h = xdst.shape
    row_spec = pl.BlockSpec((_BLK, h), lambda i: (i, 0))
    return pl.pallas_call(
        functools.partial(_layer_kernel, relu=relu),
        grid=(n // _BLK,),
        in_specs=[
            row_spec, row_spec, row_spec,
            pl.BlockSpec((_BLK, 3), lambda i: (i, 0)),
            row_spec,
            pl.BlockSpec((3, h, h), lambda i: (0, 0, 0)),
            pl.BlockSpec((h, h), lambda i: (0, 0)),
            pl.BlockSpec((1, h), lambda i: (0, 0)),
        ],
        out_specs=row_spec,
        out_shape=jax.ShapeDtypeStruct((n, h), jnp.float32),
    )(s0, s1, s2, rinv, xdst, wl, wrm, bm)


def _head_kernel(ue_ref, ie_ref, d1u_ref, d1i_ref, bd1_ref, d2_ref, bd2_ref,
                 out_ref):
    h = jnp.dot(ue_ref[...], d1u_ref[...], preferred_element_type=jnp.float32)
    h += jnp.dot(ie_ref[...], d1i_ref[...], preferred_element_type=jnp.float32)
    h += bd1_ref[...]
    h = jnp.maximum(h, 0.0)
    out = jnp.dot(h, d2_ref[...], preferred_element_type=jnp.float32)
    out_ref[...] = out + bd2_ref[...]


def _head_call(ue, ie, d1u, d1i, bd1, d2p, bd2p):
    b, h = ue.shape
    row_spec = pl.BlockSpec((_HBLK, h), lambda i: (i, 0))
    w_spec = pl.BlockSpec((h, h), lambda i: (0, 0))
    b_spec = pl.BlockSpec((1, h), lambda i: (0, 0))
    return pl.pallas_call(
        _head_kernel,
        grid=(b // _HBLK,),
        in_specs=[row_spec, row_spec, w_spec, w_spec, b_spec, w_spec, b_spec],
        out_specs=row_spec,
        out_shape=jax.ShapeDtypeStruct((b, h), jnp.float32),
    )(ue, ie, d1u, d1i, bd1, d2p, bd2p)


def _seg_sum(x_src, ei, n_dst):
    msg = jnp.take(x_src, ei[0], axis=0)
    return jax.ops.segment_sum(msg, ei[1], num_segments=n_dst)


def _rinv(edges, rels, n_dst):
    cols = []
    for r in rels:
        c = jax.ops.segment_sum(jnp.ones((edges[r].shape[1],), jnp.float32),
                                edges[r][1], num_segments=n_dst)
        cols.append(1.0 / jnp.maximum(c, 1.0))
    return jnp.stack(cols, axis=1)


def kernel(edge_view, edge_save, edge_buy, edge_viewed_by, edge_saved_by,
           edge_bought_by, user_ids, item_ids, user_table, item_table,
           W1_l, W1_r, b1, W2_l, W2_r, b2, D1, bd1, D2, bd2):
    edges = [edge_view, edge_save, edge_buy,
             edge_viewed_by, edge_saved_by, edge_bought_by]
    nu = user_table.shape[0]
    ni = item_table.shape[0]

    rinv_i = _rinv(edges, range(3), ni)
    rinv_u = _rinv(edges, range(3, 6), nu)

    def layer(xu, xi, Wl, Wr, b, relu):
        si = [_seg_sum(xu, edges[r], ni) for r in range(3)]
        su = [_seg_sum(xi, edges[r], nu) for r in range(3, 6)]
        item_out = _layer_call(si[0], si[1], si[2], rinv_i, xi,
                               Wl[0:3] / 3.0,
                               jnp.mean(Wr[0:3], axis=0),
                               jnp.mean(b[0:3], axis=0)[None, :], relu)
        user_out = _layer_call(su[0], su[1], su[2], rinv_u, xu,
                               Wl[3:6] / 3.0,
                               jnp.mean(Wr[3:6], axis=0),
                               jnp.mean(b[3:6], axis=0)[None, :], relu)
        return user_out, item_out

    xu, xi = layer(user_table, item_table, W1_l, W1_r, b1, relu=True)
    xu, xi = layer(xu, xi, W2_l, W2_r, b2, relu=False)

    ue = jnp.take(xu, user_ids, axis=0)
    ie = jnp.take(xi, item_ids, axis=0)

    hid = D1.shape[1]
    d1u = D1[:hid]
    d1i = D1[hid:]
    nout = D2.shape[1]
    d2p = jnp.zeros((hid, hid), jnp.float32).at[:, :nout].set(D2)
    bd2p = jnp.zeros((1, hid), jnp.float32).at[0, :nout].set(bd2)
    logits = _head_call(ue, ie, d1u, d1i, bd1[None, :], d2p, bd2p)
    return logits[:, :nout]
